# baseline (speedup 1.0000x reference)
import dataclasses

import jax
import jax.numpy as jnp
from jax import lax
from jax.experimental import pallas as pl
from jax.experimental.pallas import tpu as pltpu
from jax.experimental.pallas import tpu_sc as plsc

D = 256
E = 160000
NC, NS, L = 2, 16, 16
NW = NC * NS
EPW = E // NW
W = 40
NBUF = 4
NCHUNK = EPW // W


def _sc_dot_body(xu_hbm, xm_hbm, edge_hbm, out_hbm,
                 idxu_v, idxm_v, ubuf, mbuf, out_v, sems):
    wid = lax.axis_index("c") * NS + lax.axis_index("s")
    base = wid * EPW
    pltpu.sync_copy(edge_hbm.at[pl.ds(E + base, EPW)], idxu_v)
    pltpu.sync_copy(edge_hbm.at[pl.ds(base, EPW)], idxm_v)

    def start(ci, p):
        c = ci * W
        pltpu.async_copy(xu_hbm.at[idxu_v.at[pl.ds(c, W)]], ubuf.at[p],
                         sems.at[p])
        pltpu.async_copy(xm_hbm.at[idxm_v.at[pl.ds(c, W)]], mbuf.at[p],
                         sems.at[p])

    def drain(p):
        pltpu.make_async_copy(xu_hbm.at[idxu_v.at[pl.ds(0, W)]], ubuf.at[p],
                              sems.at[p]).wait()
        pltpu.make_async_copy(xm_hbm.at[idxm_v.at[pl.ds(0, W)]], mbuf.at[p],
                              sems.at[p]).wait()

    for j in range(NBUF - 1):
        start(j, j)

    @pl.loop(0, NCHUNK)
    def _chunk(ci):
        p = lax.rem(ci, NBUF)

        @pl.when(ci + NBUF - 1 < NCHUNK)
        def _prefetch():
            start(ci + NBUF - 1, lax.rem(ci + NBUF - 1, NBUF))

        drain(p)
        c = ci * W

        @pl.loop(0, (W + L - 1) // L)
        def _grp(g):
            e0 = jnp.minimum(g * L, W - L)
            lane = lax.iota(jnp.int32, L)
            r = jnp.zeros((L,), jnp.float32)
            for i in range(L):
                e = e0 + i
                acc = jnp.zeros((L,), jnp.float32)
                for k in range(D // (2 * L)):
                    au, bu = plsc.unpack(
                        plsc.bitcast(ubuf[p, e, pl.ds(k * L, L)], jnp.bfloat16),
                        format=plsc.PackFormat.INTERLEAVED)
                    am, bm = plsc.unpack(
                        plsc.bitcast(mbuf[p, e, pl.ds(k * L, L)], jnp.bfloat16),
                        format=plsc.PackFormat.INTERLEAVED)
                    acc = acc + au * am
                    acc = acc + bu * bm
                r = jnp.where(lane == i, jnp.sum(acc), r)
            out_v[pl.ds(c + e0, L)] = r

    pltpu.sync_copy(out_v, out_hbm.at[pl.ds(base, EPW)])


def _pack_bf16_words(x):
    u = lax.bitcast_convert_type(x, jnp.uint32)
    r = (u + jnp.uint32(0x7FFF) + ((u >> 16) & jnp.uint32(1))) >> 16
    return (r[:, : D // 2] | (r[:, D // 2 :] << 16)).astype(jnp.int32)


def kernel(x_user, x_movie, edge_label_index):
    mesh = plsc.VectorSubcoreMesh(core_axis_name="c", subcore_axis_name="s")
    cp = pltpu.CompilerParams()
    if "needs_layout_passes" in pltpu.CompilerParams.__dataclass_fields__:
        cp = dataclasses.replace(cp, needs_layout_passes=False)
    run = pl.kernel(
        _sc_dot_body,
        out_type=jax.ShapeDtypeStruct((E,), jnp.float32),
        mesh=mesh,
        compiler_params=cp,
        scratch_types=[
            pltpu.VMEM((EPW,), jnp.int32),
            pltpu.VMEM((EPW,), jnp.int32),
            pltpu.VMEM((NBUF, W, D // 2), jnp.int32),
            pltpu.VMEM((NBUF, W, D // 2), jnp.int32),
            pltpu.VMEM((EPW,), jnp.float32),
            pltpu.SemaphoreType.DMA((NBUF,)),
        ],
    )
    return run(_pack_bf16_words(x_user), _pack_bf16_words(x_movie),
               edge_label_index.reshape(-1))


# speedup vs baseline: 1.1799x; 1.1799x over previous
import dataclasses

import jax
import jax.numpy as jnp
from jax import lax
from jax.experimental import pallas as pl
from jax.experimental.pallas import tpu as pltpu
from jax.experimental.pallas import tpu_sc as plsc

D = 256
E = 160000
NC, NS, L = 2, 16, 16
NW = NC * NS
EPW = E // NW
W = 200
NBUF = 2
NCHUNK = EPW // W


def _sc_dot_body(xu_hbm, xm_hbm, edge_hbm, out_hbm,
                 idxu_v, idxm_v, ubuf, mbuf, out_v, sems):
    wid = lax.axis_index("c") * NS + lax.axis_index("s")
    base = wid * EPW
    pltpu.sync_copy(edge_hbm.at[pl.ds(E + base, EPW)], idxu_v)
    pltpu.sync_copy(edge_hbm.at[pl.ds(base, EPW)], idxm_v)

    def start(ci, p):
        c = ci * W
        pltpu.async_copy(xu_hbm.at[idxu_v.at[pl.ds(c, W)]], ubuf.at[p],
                         sems.at[p])
        pltpu.async_copy(xm_hbm.at[idxm_v.at[pl.ds(c, W)]], mbuf.at[p],
                         sems.at[p])

    def drain(p):
        pltpu.make_async_copy(xu_hbm.at[idxu_v.at[pl.ds(0, W)]], ubuf.at[p],
                              sems.at[p]).wait()
        pltpu.make_async_copy(xm_hbm.at[idxm_v.at[pl.ds(0, W)]], mbuf.at[p],
                              sems.at[p]).wait()

    for j in range(NBUF - 1):
        start(j, j)

    @pl.loop(0, NCHUNK)
    def _chunk(ci):
        p = lax.rem(ci, NBUF)

        @pl.when(ci + NBUF - 1 < NCHUNK)
        def _prefetch():
            start(ci + NBUF - 1, lax.rem(ci + NBUF - 1, NBUF))

        drain(p)
        c = ci * W

        @pl.loop(0, (W + L - 1) // L)
        def _grp(g):
            e0 = jnp.minimum(g * L, W - L)
            lane = lax.iota(jnp.int32, L)
            r = jnp.zeros((L,), jnp.float32)
            for i in range(L):
                e = e0 + i
                acc = jnp.zeros((L,), jnp.float32)
                for k in range(D // (2 * L)):
                    au, bu = plsc.unpack(
                        plsc.bitcast(ubuf[p, e, pl.ds(k * L, L)], jnp.bfloat16),
                        format=plsc.PackFormat.INTERLEAVED)
                    am, bm = plsc.unpack(
                        plsc.bitcast(mbuf[p, e, pl.ds(k * L, L)], jnp.bfloat16),
                        format=plsc.PackFormat.INTERLEAVED)
                    acc = acc + au * am
                    acc = acc + bu * bm
                r = jnp.where(lane == i, jnp.sum(acc), r)
            out_v[pl.ds(c + e0, L)] = r

    pltpu.sync_copy(out_v, out_hbm.at[pl.ds(base, EPW)])


def _pack_bf16_words(x):
    def rtne(v):
        u = lax.bitcast_convert_type(v, jnp.uint32)
        return (u + jnp.uint32(0x7FFF) + ((u >> 16) & jnp.uint32(1))) >> 16
    return (rtne(x[:, : D // 2]) | (rtne(x[:, D // 2 :]) << 16)).astype(
        jnp.int32)


def kernel(x_user, x_movie, edge_label_index):
    mesh = plsc.VectorSubcoreMesh(core_axis_name="c", subcore_axis_name="s")
    cp = pltpu.CompilerParams()
    if "needs_layout_passes" in pltpu.CompilerParams.__dataclass_fields__:
        cp = dataclasses.replace(cp, needs_layout_passes=False)
    run = pl.kernel(
        _sc_dot_body,
        out_type=jax.ShapeDtypeStruct((E,), jnp.float32),
        mesh=mesh,
        compiler_params=cp,
        scratch_types=[
            pltpu.VMEM((EPW,), jnp.int32),
            pltpu.VMEM((EPW,), jnp.int32),
            pltpu.VMEM((NBUF, W, D // 2), jnp.int32),
            pltpu.VMEM((NBUF, W, D // 2), jnp.int32),
            pltpu.VMEM((EPW,), jnp.float32),
            pltpu.SemaphoreType.DMA((NBUF,)),
        ],
    )
    return run(_pack_bf16_words(x_user), _pack_bf16_words(x_movie),
               edge_label_index.reshape(-1))
